# Initial kernel scaffold; baseline (speedup 1.0000x reference)
import jax
import jax.numpy as jnp
from jax.experimental import pallas as pl


def kernel(x, s):
    raise NotImplementedError("write your pallas kernel here")


# speedup vs baseline: 1.0006x; 1.0006x over previous
import jax
import jax.numpy as jnp
from jax.experimental import pallas as pl


def _copy_block(x_ref, o_ref):
    o_ref[...] = x_ref[...]


def kernel(x, s):
    del s
    b, m, n = x.shape
    xf = x.reshape(b * m, n)
    rows = b * m
    block_rows = 1024
    grid = (rows // block_rows,)
    out = pl.pallas_call(
        _copy_block,
        grid=grid,
        in_specs=[pl.BlockSpec((block_rows, n), lambda i: (i, 0))],
        out_specs=pl.BlockSpec((block_rows, n), lambda i: (i, 0)),
        out_shape=jax.ShapeDtypeStruct((rows, n), x.dtype),
    )(xf)
    return out.reshape(b, m, n)
